# baseline (speedup 1.0000x reference)
import functools

import jax
import jax.numpy as jnp
from jax.experimental import pallas as pl
from jax.experimental.pallas import tpu as pltpu

_LANE_BLOCK = 16384


def _mask_body(nsel, allowed_ref, scores_hbm, out_ref, gath_ref, sem):
    i = pl.program_id(0)
    base = i * _LANE_BLOCK
    nrows = out_ref.shape[0]

    def col_copy(k):
        src = pl.multiple_of((allowed_ref[k] // 128) * 128, 128)
        return pltpu.make_async_copy(
            scores_hbm.at[:, pl.ds(src, 128)],
            gath_ref.at[:, pl.ds(k * 128, 128)],
            sem,
        )

    @pl.when(i == 0)
    def _():
        for k in range(nsel):
            col_copy(k).start()
        for k in range(nsel):
            col_copy(k).wait()

    out_ref[...] = jnp.full(out_ref.shape, -jnp.inf, out_ref.dtype)
    lanes = jax.lax.broadcasted_iota(jnp.int32, (nrows, 128), 1)
    for k in range(nsel):
        off = allowed_ref[k] - base

        @pl.when((off >= 0) & (off < _LANE_BLOCK))
        def _():
            chunk_base = pl.multiple_of((off // 128) * 128, 128)
            chunk = out_ref[:, pl.ds(chunk_base, 128)]
            gath = gath_ref[:, k * 128 : (k + 1) * 128]
            chunk = jnp.where(lanes == off % 128, gath, chunk)
            out_ref[:, pl.ds(chunk_base, 128)] = chunk


def kernel(input_ids, scores, allowed):
    nrows, ncols = scores.shape
    nsel = allowed.shape[0]
    grid = pl.cdiv(ncols, _LANE_BLOCK)
    grid_spec = pltpu.PrefetchScalarGridSpec(
        num_scalar_prefetch=1,
        grid=(grid,),
        in_specs=[pl.BlockSpec(memory_space=pl.ANY)],
        out_specs=pl.BlockSpec((nrows, _LANE_BLOCK), lambda i, a: (0, i)),
        scratch_shapes=[
            pltpu.VMEM((nrows, nsel * 128), jnp.float32),
            pltpu.SemaphoreType.DMA,
        ],
    )
    return pl.pallas_call(
        functools.partial(_mask_body, nsel),
        grid_spec=grid_spec,
        out_shape=jax.ShapeDtypeStruct(scores.shape, scores.dtype),
    )(allowed, scores)


# speedup vs baseline: 1.1124x; 1.1124x over previous
import functools

import jax
import jax.numpy as jnp
from jax.experimental import pallas as pl
from jax.experimental.pallas import tpu as pltpu

_LANE_BLOCK = 76928


def _mask_body(nsel, allowed_ref, scores_hbm, out_ref, gath_ref, sem):
    i = pl.program_id(0)
    base = i * _LANE_BLOCK
    nrows = out_ref.shape[0]

    def col_copy(k):
        src = pl.multiple_of((allowed_ref[k] // 128) * 128, 128)
        return pltpu.make_async_copy(
            scores_hbm.at[:, pl.ds(src, 128)],
            gath_ref.at[:, pl.ds(k * 128, 128)],
            sem,
        )

    @pl.when(i == 0)
    def _():
        for k in range(nsel):
            col_copy(k).start()
        for k in range(nsel):
            col_copy(k).wait()

    out_ref[...] = jnp.full(out_ref.shape, -jnp.inf, out_ref.dtype)
    lanes = jax.lax.broadcasted_iota(jnp.int32, (nrows, 128), 1)
    for k in range(nsel):
        off = allowed_ref[k] - base

        @pl.when((off >= 0) & (off < _LANE_BLOCK))
        def _():
            chunk_base = pl.multiple_of((off // 128) * 128, 128)
            chunk = out_ref[:, pl.ds(chunk_base, 128)]
            gath = gath_ref[:, k * 128 : (k + 1) * 128]
            chunk = jnp.where(lanes == off % 128, gath, chunk)
            out_ref[:, pl.ds(chunk_base, 128)] = chunk


def kernel(input_ids, scores, allowed):
    nrows, ncols = scores.shape
    nsel = allowed.shape[0]
    grid = pl.cdiv(ncols, _LANE_BLOCK)
    grid_spec = pltpu.PrefetchScalarGridSpec(
        num_scalar_prefetch=1,
        grid=(grid,),
        in_specs=[pl.BlockSpec(memory_space=pl.ANY)],
        out_specs=pl.BlockSpec((nrows, _LANE_BLOCK), lambda i, a: (0, i)),
        scratch_shapes=[
            pltpu.VMEM((nrows, nsel * 128), jnp.float32),
            pltpu.SemaphoreType.DMA,
        ],
    )
    return pl.pallas_call(
        functools.partial(_mask_body, nsel),
        grid_spec=grid_spec,
        out_shape=jax.ShapeDtypeStruct(scores.shape, scores.dtype),
    )(allowed, scores)


# speedup vs baseline: 1.1250x; 1.0113x over previous
import functools

import jax
import jax.numpy as jnp
from jax.experimental import pallas as pl
from jax.experimental.pallas import tpu as pltpu

_LANE_BLOCK = 38528


def _mask_body(nsel, allowed_ref, scores_hbm, out_ref, gath_ref, sem):
    i = pl.program_id(0)
    base = i * _LANE_BLOCK
    nrows = out_ref.shape[0]

    def col_copy(k):
        src = pl.multiple_of((allowed_ref[k] // 128) * 128, 128)
        return pltpu.make_async_copy(
            scores_hbm.at[:, pl.ds(src, 128)],
            gath_ref.at[:, pl.ds(k * 128, 128)],
            sem,
        )

    @pl.when(i == 0)
    def _():
        for k in range(nsel):
            col_copy(k).start()
        for k in range(nsel):
            col_copy(k).wait()

    out_ref[...] = jnp.full(out_ref.shape, -jnp.inf, out_ref.dtype)
    lanes = jax.lax.broadcasted_iota(jnp.int32, (nrows, 128), 1)
    for k in range(nsel):
        off = allowed_ref[k] - base

        @pl.when((off >= 0) & (off < _LANE_BLOCK))
        def _():
            chunk_base = pl.multiple_of((off // 128) * 128, 128)
            chunk = out_ref[:, pl.ds(chunk_base, 128)]
            gath = gath_ref[:, k * 128 : (k + 1) * 128]
            chunk = jnp.where(lanes == off % 128, gath, chunk)
            out_ref[:, pl.ds(chunk_base, 128)] = chunk


def kernel(input_ids, scores, allowed):
    nrows, ncols = scores.shape
    nsel = allowed.shape[0]
    grid = pl.cdiv(ncols, _LANE_BLOCK)
    grid_spec = pltpu.PrefetchScalarGridSpec(
        num_scalar_prefetch=1,
        grid=(grid,),
        in_specs=[pl.BlockSpec(memory_space=pl.ANY)],
        out_specs=pl.BlockSpec((nrows, _LANE_BLOCK), lambda i, a: (0, i)),
        scratch_shapes=[
            pltpu.VMEM((nrows, nsel * 128), jnp.float32),
            pltpu.SemaphoreType.DMA,
        ],
    )
    return pl.pallas_call(
        functools.partial(_mask_body, nsel),
        grid_spec=grid_spec,
        out_shape=jax.ShapeDtypeStruct(scores.shape, scores.dtype),
    )(allowed, scores)


# speedup vs baseline: 1.1413x; 1.0145x over previous
import functools

import jax
import jax.numpy as jnp
from jax.experimental import pallas as pl
from jax.experimental.pallas import tpu as pltpu

_LANE_BLOCK = 32768


def _mask_body(nsel, allowed_ref, scores_hbm, out_ref, gath_ref, sem):
    i = pl.program_id(0)
    base = i * _LANE_BLOCK
    nrows = out_ref.shape[0]

    def col_copy(k):
        src = pl.multiple_of((allowed_ref[k] // 128) * 128, 128)
        return pltpu.make_async_copy(
            scores_hbm.at[:, pl.ds(src, 128)],
            gath_ref.at[:, pl.ds(k * 128, 128)],
            sem,
        )

    @pl.when(i == 0)
    def _():
        for k in range(nsel):
            col_copy(k).start()

    out_ref[...] = jnp.full(out_ref.shape, -jnp.inf, out_ref.dtype)

    @pl.when(i == 0)
    def _():
        for k in range(nsel):
            col_copy(k).wait()

    lanes = jax.lax.broadcasted_iota(jnp.int32, (nrows, 128), 1)
    for k in range(nsel):
        off = allowed_ref[k] - base

        @pl.when((off >= 0) & (off < _LANE_BLOCK))
        def _():
            chunk_base = pl.multiple_of((off // 128) * 128, 128)
            chunk = out_ref[:, pl.ds(chunk_base, 128)]
            gath = gath_ref[:, k * 128 : (k + 1) * 128]
            chunk = jnp.where(lanes == off % 128, gath, chunk)
            out_ref[:, pl.ds(chunk_base, 128)] = chunk


def kernel(input_ids, scores, allowed):
    nrows, ncols = scores.shape
    nsel = allowed.shape[0]
    grid = pl.cdiv(ncols, _LANE_BLOCK)
    grid_spec = pltpu.PrefetchScalarGridSpec(
        num_scalar_prefetch=1,
        grid=(grid,),
        in_specs=[pl.BlockSpec(memory_space=pl.ANY)],
        out_specs=pl.BlockSpec((nrows, _LANE_BLOCK), lambda i, a: (0, i)),
        scratch_shapes=[
            pltpu.VMEM((nrows, nsel * 128), jnp.float32),
            pltpu.SemaphoreType.DMA,
        ],
    )
    return pl.pallas_call(
        functools.partial(_mask_body, nsel),
        grid_spec=grid_spec,
        out_shape=jax.ShapeDtypeStruct(scores.shape, scores.dtype),
    )(allowed, scores)
